# baseline (speedup 1.0000x reference)
import functools

import jax
import jax.numpy as jnp
from jax import lax
from jax.experimental import pallas as pl
from jax.experimental.pallas import tpu as pltpu
from jax.experimental.pallas import tpu_sc as plsc

B, L, D = 4096, 50, 128
NC, NS = 2, 16
NW = NC * NS
PER_W = B // NW
NBUF = 2
NGROUP = L // NBUF


@functools.partial(
    pl.kernel,
    mesh=plsc.VectorSubcoreMesh(core_axis_name="c", subcore_axis_name="s"),
    out_type=jax.ShapeDtypeStruct((L, B, D), jnp.float32),
    scratch_types=[
        pltpu.VMEM((L, 8, PER_W), jnp.int32),
        pltpu.VMEM((2 * NBUF, PER_W, D), jnp.float32),
    ]
    + [pltpu.SemaphoreType.DMA] * (2 * NBUF),
)
def _gather_kernel(idx_hbm, table_hbm, out_hbm, idx_v, bufs, *sems):
    gsems = sems[:NBUF]
    ssems = sems[NBUF:]
    wid = lax.axis_index("s") * NC + lax.axis_index("c")
    base = wid * PER_W
    group8 = (wid // 8) * 8
    lane = lax.rem(wid, 8)
    pltpu.sync_copy(idx_hbm.at[:, pl.ds(group8, 8)], idx_v)

    def my_idx(j):
        return idx_v.at[j].at[lane]

    for b in range(NBUF):
        pltpu.async_copy(table_hbm.at[my_idx(b)], bufs.at[b], gsems[b])

    def body(g, carry):
        p = lax.rem(g, 2)
        pn = 1 - p
        for b in range(NBUF):
            j = g * NBUF + b
            cur = p * NBUF + b
            nxt = pn * NBUF + b
            pltpu.make_async_copy(
                table_hbm.at[my_idx(j)], bufs.at[cur], gsems[b]
            ).wait()

            @pl.when(g > 0)
            def _drain():
                pltpu.make_async_copy(
                    bufs.at[nxt], out_hbm.at[j].at[pl.ds(base, PER_W)], ssems[b]
                ).wait()

            pltpu.async_copy(
                bufs.at[cur], out_hbm.at[j].at[pl.ds(base, PER_W)], ssems[b]
            )

            @pl.when(g < NGROUP - 1)
            def _next_gather():
                pltpu.async_copy(
                    table_hbm.at[my_idx(j + NBUF)], bufs.at[nxt], gsems[b]
                )

        return carry

    lax.fori_loop(0, NGROUP, body, 0)

    for b in range(NBUF):
        pltpu.make_async_copy(
            bufs.at[b], out_hbm.at[0].at[pl.ds(base, PER_W)], ssems[b]
        ).wait()


def kernel(token_ids, weight):
    idx = token_ids.astype(jnp.int32).T.reshape(L, NW, PER_W)
    out = _gather_kernel(idx, weight)
    return out.transpose(1, 0, 2)


# speedup vs baseline: 1.0241x; 1.0241x over previous
import functools

import jax
import jax.numpy as jnp
from jax import lax
from jax.experimental import pallas as pl
from jax.experimental.pallas import tpu as pltpu
from jax.experimental.pallas import tpu_sc as plsc

B, L, D = 4096, 50, 128
NC, NS = 2, 16
NW = NC * NS
PER_W = B // NW
NBUF = 2
NGROUP = L // NBUF


@functools.partial(
    pl.kernel,
    mesh=plsc.VectorSubcoreMesh(core_axis_name="c", subcore_axis_name="s"),
    out_type=jax.ShapeDtypeStruct((L, B, D), jnp.float32),
    scratch_types=[
        pltpu.VMEM((L, PER_W), jnp.int32),
        pltpu.VMEM((2 * NBUF, PER_W, D), jnp.float32),
    ]
    + [pltpu.SemaphoreType.DMA] * (2 * NBUF),
)
def _gather_kernel(idx_hbm, table_hbm, out_hbm, idx_v, bufs, *sems):
    gsems = sems[:NBUF]
    ssems = sems[NBUF:]
    wid = lax.axis_index("s") * NC + lax.axis_index("c")
    base = wid * PER_W
    pltpu.sync_copy(idx_hbm.at[wid], idx_v)

    for b in range(NBUF):
        pltpu.async_copy(table_hbm.at[idx_v.at[b]], bufs.at[b], gsems[b])

    def body(g, carry):
        p = lax.rem(g, 2)
        pn = 1 - p
        for b in range(NBUF):
            j = g * NBUF + b
            cur = p * NBUF + b
            nxt = pn * NBUF + b
            pltpu.make_async_copy(
                table_hbm.at[idx_v.at[j]], bufs.at[cur], gsems[b]
            ).wait()

            @pl.when(g > 0)
            def _drain():
                pltpu.make_async_copy(
                    bufs.at[nxt], out_hbm.at[j].at[pl.ds(base, PER_W)], ssems[b]
                ).wait()

            pltpu.async_copy(
                bufs.at[cur], out_hbm.at[j].at[pl.ds(base, PER_W)], ssems[b]
            )

            @pl.when(g < NGROUP - 1)
            def _next_gather():
                pltpu.async_copy(
                    table_hbm.at[idx_v.at[j + NBUF]], bufs.at[nxt], gsems[b]
                )

        return carry

    lax.fori_loop(0, NGROUP, body, 0)

    for b in range(NBUF):
        pltpu.make_async_copy(
            bufs.at[b], out_hbm.at[0].at[pl.ds(base, PER_W)], ssems[b]
        ).wait()


def kernel(token_ids, weight):
    idx = token_ids.astype(jnp.int32).reshape(NW, PER_W, L).transpose(0, 2, 1)
    out = _gather_kernel(idx, weight)
    return out.transpose(1, 0, 2)
